# baseline (speedup 1.0000x reference)
import functools

import jax
import jax.numpy as jnp
from jax import lax
from jax.experimental import pallas as pl
from jax.experimental.pallas import tpu as pltpu
from jax.experimental.pallas import tpu_sc as plsc

NN = 10000
EE = 160000
GG = 64
OUT = 128

NPAD = 10240
NTILES = 32
EPT = 5120
EPAD = EPT * NTILES
KB = 256
NB = EPT // KB
FC = 64
RPT = NPAD // 16
MB = 512

def _mesh():
  return plsc.VectorSubcoreMesh(
      core_axis_name="c", subcore_axis_name="s", num_cores=2, num_subcores=16
  )


@functools.cache
def _make_spmm(nchunk, fc):
  out_t = jax.ShapeDtypeStruct((2, NPAD, nchunk, fc), jnp.float32)
  R = 4
  scratch = [
      pltpu.VMEM((EPT,), jnp.int32),
      pltpu.VMEM((NB, KB), jnp.int32),
      pltpu.VMEM((EPT,), jnp.float32),
      pltpu.VMEM((EPT,), jnp.int32),
      pltpu.VMEM((R, KB, fc), jnp.float32),
      pltpu.VMEM((R, KB, fc), jnp.float32),
      pltpu.VMEM((KB, fc), jnp.float32),
      pltpu.VMEM_SHARED((NPAD, fc), jnp.float32),
      pltpu.SemaphoreType.DMA((R,)),
      pltpu.SemaphoreType.DMA((R,)),
  ]
  NG = NB // R

  @functools.partial(
      pl.kernel, out_type=out_t, mesh=_mesh(), scratch_types=scratch,
      compiler_params=pltpu.CompilerParams(use_tc_tiling_on_sc=False))
  def spmm(row_hbm, col_hbm, ew_hbm, y_hbm, z_hbm,
           rowb, colb, ewb, rowadj, gbufs, sbufs, zbuf, acc, gsem, ssem):
    core = lax.axis_index("c")
    sid = lax.axis_index("s")
    wid = sid * 2 + core
    pltpu.sync_copy(row_hbm.at[wid], rowb)
    pltpu.sync_copy(col_hbm.at[wid], colb)
    pltpu.sync_copy(ew_hbm.at[wid], ewb)

    def zero_zbuf(i, _):
      for v in range(fc // 16):
        zbuf[i, pl.ds(v * 16, 16)] = jnp.zeros((16,), jnp.float32)
      return 0
    lax.fori_loop(0, KB, zero_zbuf, 0)

    def issue_gather(b, r):
      pltpu.async_copy(
          y_hbm.at[rowadj.at[pl.ds(b * KB, KB)]], gbufs.at[r], gsem.at[r])

    def wait_gather(r):
      pltpu.make_async_copy(
          y_hbm.at[pl.ds(0, KB)], gbufs.at[r], gsem.at[r]).wait()

    def wait_scatter(r):
      pltpu.make_async_copy(
          y_hbm.at[pl.ds(0, KB)], sbufs.at[r], ssem.at[r]).wait()

    def chunk_body(c, _):
      def adj(i, _):
        rowadj[pl.ds(i * 16, 16)] = rowb[pl.ds(i * 16, 16)] + c
        return 0
      lax.fori_loop(0, EPT // 16, adj, 0)
      for i in range(RPT // KB):
        pltpu.sync_copy(zbuf, acc.at[pl.ds(sid * RPT + i * KB, KB)])
      plsc.subcore_barrier()

      for r in range(R):
        issue_gather(r, r)

      def group_body(i, _):
        for r in range(R):
          b = i * R + r
          wait_gather(r)

          @pl.when(i > 0)
          def _():
            wait_scatter(r)

          def scale(s, _):
            ew16 = ewb[pl.ds(b * KB + s * 16, 16)]
            for j in range(16):
              w = ew16[j]
              e = s * 16 + j
              for v in range(fc // 16):
                sbufs[r, e, pl.ds(v * 16, 16)] = (
                    gbufs[r, e, pl.ds(v * 16, 16)] * w)
            return 0
          lax.fori_loop(0, KB // 16, scale, 0)
          pltpu.async_copy(
              sbufs.at[r], acc.at[colb.at[b]], ssem.at[r], add=True)

          @pl.when(i < NG - 1)
          def _():
            issue_gather(b + R, r)
        return 0
      lax.fori_loop(0, NG, group_body, 0)
      for r in range(R):
        wait_scatter(r)
      plsc.subcore_barrier()
      pltpu.sync_copy(acc.at[pl.ds(sid * RPT, RPT)],
                      z_hbm.at[core, pl.ds(sid * RPT, RPT), c])
      return 0
    lax.fori_loop(0, nchunk, chunk_body, 0)

  return spmm


def _spmm_apply(nchunk, fc, row_t, col_t, ew_t, ytab):
  return _make_spmm(nchunk, fc)(row_t, col_t, ew_t, ytab)


@functools.cache
def _make_deg():
  out_t = jax.ShapeDtypeStruct((NTILES, NPAD), jnp.float32)
  scratch = [
      pltpu.VMEM((EPT,), jnp.int32),
      pltpu.VMEM((EPT,), jnp.float32),
      pltpu.VMEM((NPAD,), jnp.float32),
  ]

  @functools.partial(
      pl.kernel, out_type=out_t, mesh=_mesh(), scratch_types=scratch,
      compiler_params=pltpu.CompilerParams(
          use_tc_tiling_on_sc=False, needs_layout_passes=False))
  def deg(col_hbm, ew_hbm, out_hbm, colb, ewb, dacc):
    core = lax.axis_index("c")
    sid = lax.axis_index("s")
    wid = sid * 2 + core
    pltpu.sync_copy(col_hbm.at[wid], colb)
    pltpu.sync_copy(ew_hbm.at[wid], ewb)

    def zero(i, _):
      dacc[pl.ds(i * 16, 16)] = jnp.zeros((16,), jnp.float32)
      return 0
    lax.fori_loop(0, NPAD // 16, zero, 0)

    def accum(i, _):
      col16 = colb[pl.ds(i * 16, 16)]
      ew16 = ewb[pl.ds(i * 16, 16)]
      plsc.addupdate_scatter(dacc, [col16], ew16)
      return 0
    lax.fori_loop(0, EPT // 16, accum, 0)
    pltpu.sync_copy(dacc, out_hbm.at[wid])

  return deg


def _compute_dinv(degpt):
  def body(p_ref, o_ref):
    i = pl.program_id(0)
    rows = i * MB + lax.broadcasted_iota(jnp.int32, (MB, 1), 0)
    deg = jnp.sum(p_ref[...], axis=1, keepdims=True) + 1.0
    o_ref[...] = jnp.where(rows < NN, lax.rsqrt(deg), 0.0)

  return pl.pallas_call(
      body, grid=(NPAD // MB,),
      in_specs=[pl.BlockSpec((MB, NTILES), lambda i: (i, 0))],
      out_specs=pl.BlockSpec((MB, 1), lambda i: (i, 0)),
      out_shape=jax.ShapeDtypeStruct((NPAD, 1), jnp.float32),
  )(degpt)


def _mm_y(a, w, dinv2, stats=None, gamma=None, beta=None):
  m_, k_ = a.shape
  f_ = w.shape[1]
  nbk = min(512, f_)
  normalize = stats is not None

  def body(*refs):
    if normalize:
      a_ref, w_ref, d_ref, s_ref, g_ref, be_ref, o_ref = refs
    else:
      a_ref, w_ref, d_ref, o_ref = refs
    aa = a_ref[...]
    if normalize:
      s = s_ref[...]
      mu = s[0:1, :] * (1.0 / NN)
      var = s[1:2, :] * (1.0 / NN) - mu * mu
      aa = jnp.maximum(
          (aa - mu) * lax.rsqrt(var + 1e-5) * g_ref[...] + be_ref[...], 0.0)
    y = jnp.dot(aa, w_ref[...], preferred_element_type=jnp.float32)
    o_ref[...] = y * d_ref[...]

  in_specs = [
      pl.BlockSpec((MB, k_), lambda i, j: (i, 0)),
      pl.BlockSpec((k_, nbk), lambda i, j: (0, j)),
      pl.BlockSpec((MB, 1), lambda i, j: (i, 0)),
  ]
  args = [a, w, dinv2]
  if normalize:
    in_specs += [
        pl.BlockSpec((8, k_), lambda i, j: (0, 0)),
        pl.BlockSpec((1, k_), lambda i, j: (0, 0)),
        pl.BlockSpec((1, k_), lambda i, j: (0, 0)),
    ]
    args += [stats, gamma[None, :], beta[None, :]]
  return pl.pallas_call(
      body, grid=(m_ // MB, f_ // nbk), in_specs=in_specs,
      out_specs=pl.BlockSpec((MB, nbk), lambda i, j: (i, j)),
      out_shape=jax.ShapeDtypeStruct((m_, f_), jnp.float32),
  )(*args)


def _t_stats(z, y, dinv2, b):
  f_ = y.shape[1]

  def body(z0_ref, z1_ref, y_ref, d_ref, b_ref, t_ref, s_ref):
    i = pl.program_id(0)
    t = (z0_ref[0] + z1_ref[0] + y_ref[...]) * d_ref[...] + b_ref[...]
    t_ref[...] = t
    rows = i * MB + lax.broadcasted_iota(jnp.int32, (MB, 1), 0)
    tm = jnp.where(rows < NN, t, 0.0)

    @pl.when(i == 0)
    def _():
      s_ref[...] = jnp.zeros_like(s_ref)
    s_ref[0:1, :] += jnp.sum(tm, axis=0, keepdims=True)
    s_ref[1:2, :] += jnp.sum(tm * tm, axis=0, keepdims=True)

  return pl.pallas_call(
      body, grid=(NPAD // MB,),
      in_specs=[
          pl.BlockSpec((1, MB, f_), lambda i: (0, i, 0)),
          pl.BlockSpec((1, MB, f_), lambda i: (1, i, 0)),
          pl.BlockSpec((MB, f_), lambda i: (i, 0)),
          pl.BlockSpec((MB, 1), lambda i: (i, 0)),
          pl.BlockSpec((1, f_), lambda i: (0, 0)),
      ],
      out_specs=[
          pl.BlockSpec((MB, f_), lambda i: (i, 0)),
          pl.BlockSpec((8, f_), lambda i: (0, 0)),
      ],
      out_shape=[
          jax.ShapeDtypeStruct((NPAD, f_), jnp.float32),
          jax.ShapeDtypeStruct((8, f_), jnp.float32),
      ],
  )(z, z, y, dinv2, b[None, :])


def _h3_final(z, y, dinv2, b):
  f_ = y.shape[1]

  def body(z0_ref, z1_ref, y_ref, d_ref, b_ref, o_ref):
    t = (z0_ref[0] + z1_ref[0] + y_ref[...]) * d_ref[...] + b_ref[...]
    o_ref[...] = jnp.maximum(t, 0.0)

  return pl.pallas_call(
      body, grid=(NPAD // MB,),
      in_specs=[
          pl.BlockSpec((1, MB, f_), lambda i: (0, i, 0)),
          pl.BlockSpec((1, MB, f_), lambda i: (1, i, 0)),
          pl.BlockSpec((MB, f_), lambda i: (i, 0)),
          pl.BlockSpec((MB, 1), lambda i: (i, 0)),
          pl.BlockSpec((1, f_), lambda i: (0, 0)),
      ],
      out_specs=pl.BlockSpec((MB, f_), lambda i: (i, 0)),
      out_shape=jax.ShapeDtypeStruct((NPAD, f_), jnp.float32),
  )(z, z, y, dinv2, b[None, :])


def _pool_fc(h3, batch2d, wfc, bfc):
  f_ = h3.shape[1]
  nsteps = NPAD // MB

  def body(h_ref, bt_ref, w_ref, b_ref, o_ref, sums_ref, cnt_ref):
    i = pl.program_id(0)

    @pl.when(i == 0)
    def _():
      sums_ref[...] = jnp.zeros_like(sums_ref)
      cnt_ref[...] = jnp.zeros_like(cnt_ref)
    oh = (bt_ref[...] == lax.broadcasted_iota(jnp.int32, (1, GG), 1)
          ).astype(jnp.float32)
    sums_ref[...] += lax.dot_general(
        oh, h_ref[...], (((0,), (0,)), ((), ())),
        preferred_element_type=jnp.float32)
    cnt_ref[...] += lax.dot_general(
        oh, jnp.ones((MB, 128), jnp.float32), (((0,), (0,)), ((), ())),
        preferred_element_type=jnp.float32)

    @pl.when(i == nsteps - 1)
    def _():
      pooled = sums_ref[...] / jnp.maximum(cnt_ref[:, 0:1], 1.0)
      o_ref[...] = jnp.dot(
          pooled, w_ref[...], preferred_element_type=jnp.float32) + b_ref[...]

  return pl.pallas_call(
      body, grid=(nsteps,),
      in_specs=[
          pl.BlockSpec((MB, f_), lambda i: (i, 0)),
          pl.BlockSpec((MB, 1), lambda i: (i, 0)),
          pl.BlockSpec((f_, OUT), lambda i: (0, 0)),
          pl.BlockSpec((1, OUT), lambda i: (0, 0)),
      ],
      out_specs=pl.BlockSpec((GG, OUT), lambda i: (0, 0)),
      out_shape=jax.ShapeDtypeStruct((GG, OUT), jnp.float32),
      scratch_shapes=[
          pltpu.VMEM((GG, f_), jnp.float32),
          pltpu.VMEM((GG, 128), jnp.float32),
      ],
  )(h3, batch2d, wfc, bfc[None, :])


def kernel(x, edge_index, edge_attr, batch, W1, b1, g1, be1,
           W2, b2, g2, be2, W3, b3, Wfc, bfc):
  x_pad = jnp.pad(x, ((0, NPAD - NN), (0, 0)))
  batch2d = jnp.pad(batch, (0, NPAD - NN), constant_values=GG)[:, None]
  rowp = jnp.pad(edge_index[0], (0, EPAD - EE))
  colp = jnp.pad(edge_index[1], (0, EPAD - EE))
  ewp = jnp.pad(edge_attr, (0, EPAD - EE))
  col_t = colp.reshape(NTILES, NB, KB)
  col_f = colp.reshape(NTILES, EPT)
  ew_t = ewp.reshape(NTILES, EPT)
  row32 = (rowp * 32).reshape(NTILES, EPT)
  row16 = (rowp * 16).reshape(NTILES, EPT)

  degp = _make_deg()(col_f, ew_t)
  dinv2 = _compute_dinv(degp.T)

  y1 = _mm_y(x_pad, W1, dinv2)
  z1 = _spmm_apply(32, 32, row32, col_t, ew_t, y1.reshape(NPAD * 32, 32))
  t1, s1 = _t_stats(z1.reshape(2, NPAD, 1024), y1, dinv2, b1)

  y2 = _mm_y(t1, W2, dinv2, s1, g1, be1)
  z2 = _spmm_apply(32, 32, row32, col_t, ew_t, y2.reshape(NPAD * 32, 32))
  t2, s2 = _t_stats(z2.reshape(2, NPAD, 1024), y2, dinv2, b2)

  y3 = _mm_y(t2, W3, dinv2, s2, g2, be2)
  z3 = _spmm_apply(16, 32, row16, col_t, ew_t, y3.reshape(NPAD * 16, 32))
  h3 = _h3_final(z3.reshape(2, NPAD, 512), y3, dinv2, b3)

  return _pool_fc(h3, batch2d, Wfc, bfc)


# speedup vs baseline: 1.1617x; 1.1617x over previous
import functools

import jax
import jax.numpy as jnp
from jax import lax
from jax.experimental import pallas as pl
from jax.experimental.pallas import tpu as pltpu
from jax.experimental.pallas import tpu_sc as plsc

NN = 10000
EE = 160000
GG = 64
OUT = 128

NPAD = 10240
NTILES = 32
EPT = 5120
EPAD = EPT * NTILES
KB = 128
NB = EPT // KB
FC = 64
RPT = NPAD // 16
MB = 512

def _mesh():
  return plsc.VectorSubcoreMesh(
      core_axis_name="c", subcore_axis_name="s", num_cores=2, num_subcores=16
  )


@functools.cache
def _make_spmm(nchunk, fc):
  out_t = jax.ShapeDtypeStruct((2, NPAD, nchunk * fc), jnp.float32)
  R = 4
  scratch = [
      pltpu.VMEM((EPT,), jnp.int32),
      pltpu.VMEM((NB, KB), jnp.int32),
      pltpu.VMEM((EPT,), jnp.float32),
      pltpu.VMEM((EPT,), jnp.int32),
      pltpu.VMEM((R, KB, fc), jnp.float32),
      pltpu.VMEM((R, KB, fc), jnp.float32),
      pltpu.VMEM((KB, fc), jnp.float32),
      pltpu.VMEM_SHARED((NPAD, fc), jnp.float32),
      pltpu.SemaphoreType.DMA((R,)),
      pltpu.SemaphoreType.DMA((R,)),
  ]
  NG = NB // R

  @functools.partial(
      pl.kernel, out_type=out_t, mesh=_mesh(), scratch_types=scratch,
      compiler_params=pltpu.CompilerParams(use_tc_tiling_on_sc=False))
  def spmm(row_hbm, col_hbm, ew_hbm, y_hbm, z_hbm,
           rowb, colb, ewb, rowadj, gbufs, sbufs, zbuf, acc, gsem, ssem):
    core = lax.axis_index("c")
    sid = lax.axis_index("s")
    wid = sid * 2 + core
    pltpu.sync_copy(row_hbm.at[wid], rowb)
    pltpu.sync_copy(col_hbm.at[wid], colb)
    pltpu.sync_copy(ew_hbm.at[wid], ewb)

    def zero_zbuf(i, _):
      for v in range(fc // 16):
        zbuf[i, pl.ds(v * 16, 16)] = jnp.zeros((16,), jnp.float32)
      return 0
    lax.fori_loop(0, KB, zero_zbuf, 0)

    def issue_gather(b, r):
      pltpu.async_copy(
          y_hbm.at[rowadj.at[pl.ds(b * KB, KB)]], gbufs.at[r], gsem.at[r])

    def wait_gather(r):
      pltpu.make_async_copy(
          y_hbm.at[pl.ds(0, KB)], gbufs.at[r], gsem.at[r]).wait()

    def wait_scatter(r):
      pltpu.make_async_copy(
          y_hbm.at[pl.ds(0, KB)], sbufs.at[r], ssem.at[r]).wait()

    def chunk_body(c, _):
      def adj(i, _):
        rowadj[pl.ds(i * 16, 16)] = rowb[pl.ds(i * 16, 16)] + c
        return 0
      lax.fori_loop(0, EPT // 16, adj, 0)
      for i in range(RPT // KB):
        pltpu.sync_copy(zbuf, acc.at[pl.ds(sid * RPT + i * KB, KB)])
      plsc.subcore_barrier()

      for r in range(R):
        issue_gather(r, r)

      def group_body(i, _):
        for r in range(R):
          b = i * R + r
          wait_gather(r)

          @pl.when(i > 0)
          def _():
            wait_scatter(r)

          def scale(s, _):
            ew16 = ewb[pl.ds(b * KB + s * 16, 16)]
            for j in range(16):
              w = ew16[j]
              e = s * 16 + j
              for v in range(fc // 16):
                sbufs[r, e, pl.ds(v * 16, 16)] = (
                    gbufs[r, e, pl.ds(v * 16, 16)] * w)
            return 0
          lax.fori_loop(0, KB // 16, scale, 0)
          pltpu.async_copy(
              sbufs.at[r], acc.at[colb.at[b]], ssem.at[r], add=True)

          @pl.when(i < NG - 1)
          def _():
            issue_gather(b + R, r)
        return 0
      lax.fori_loop(0, NG, group_body, 0)
      for r in range(R):
        wait_scatter(r)
      plsc.subcore_barrier()
      pltpu.sync_copy(acc.at[pl.ds(sid * RPT, RPT)],
                      z_hbm.at[core, pl.ds(sid * RPT, RPT),
                               pl.ds(c * fc, fc)])
      return 0
    lax.fori_loop(0, nchunk, chunk_body, 0)

  return spmm


def _spmm_apply(nchunk, fc, row_t, col_t, ew_t, ytab):
  return _make_spmm(nchunk, fc)(row_t, col_t, ew_t, ytab)


@functools.cache
def _make_deg():
  out_t = jax.ShapeDtypeStruct((NTILES, NPAD), jnp.float32)
  scratch = [
      pltpu.VMEM((EPT,), jnp.int32),
      pltpu.VMEM((EPT,), jnp.float32),
      pltpu.VMEM((NPAD,), jnp.float32),
  ]

  @functools.partial(
      pl.kernel, out_type=out_t, mesh=_mesh(), scratch_types=scratch,
      compiler_params=pltpu.CompilerParams(
          use_tc_tiling_on_sc=False, needs_layout_passes=False))
  def deg(col_hbm, ew_hbm, out_hbm, colb, ewb, dacc):
    core = lax.axis_index("c")
    sid = lax.axis_index("s")
    wid = sid * 2 + core
    pltpu.sync_copy(col_hbm.at[wid], colb)
    pltpu.sync_copy(ew_hbm.at[wid], ewb)

    def zero(i, _):
      dacc[pl.ds(i * 16, 16)] = jnp.zeros((16,), jnp.float32)
      return 0
    lax.fori_loop(0, NPAD // 16, zero, 0)

    def accum(i, _):
      col16 = colb[pl.ds(i * 16, 16)]
      ew16 = ewb[pl.ds(i * 16, 16)]
      plsc.addupdate_scatter(dacc, [col16], ew16)
      return 0
    lax.fori_loop(0, EPT // 16, accum, 0)
    pltpu.sync_copy(dacc, out_hbm.at[wid])

  return deg


def _compute_dinv(degpt):
  def body(p_ref, o_ref):
    i = pl.program_id(0)
    rows = i * MB + lax.broadcasted_iota(jnp.int32, (MB, 1), 0)
    deg = jnp.sum(p_ref[...], axis=1, keepdims=True) + 1.0
    o_ref[...] = jnp.where(rows < NN, lax.rsqrt(deg), 0.0)

  return pl.pallas_call(
      body, grid=(NPAD // MB,),
      in_specs=[pl.BlockSpec((MB, NTILES), lambda i: (i, 0))],
      out_specs=pl.BlockSpec((MB, 1), lambda i: (i, 0)),
      out_shape=jax.ShapeDtypeStruct((NPAD, 1), jnp.float32),
  )(degpt)


def _mm_y(a, w, dinv2, stats=None, gamma=None, beta=None):
  m_, k_ = a.shape
  f_ = w.shape[1]
  nbk = min(512, f_)
  normalize = stats is not None

  def body(*refs):
    if normalize:
      a_ref, w_ref, d_ref, s_ref, g_ref, be_ref, o_ref = refs
    else:
      a_ref, w_ref, d_ref, o_ref = refs
    aa = a_ref[...]
    if normalize:
      s = s_ref[...]
      mu = s[0:1, :] * (1.0 / NN)
      var = s[1:2, :] * (1.0 / NN) - mu * mu
      aa = jnp.maximum(
          (aa - mu) * lax.rsqrt(var + 1e-5) * g_ref[...] + be_ref[...], 0.0)
    y = jnp.dot(aa, w_ref[...], preferred_element_type=jnp.float32)
    o_ref[...] = y * d_ref[...]

  in_specs = [
      pl.BlockSpec((MB, k_), lambda i, j: (i, 0)),
      pl.BlockSpec((k_, nbk), lambda i, j: (0, j)),
      pl.BlockSpec((MB, 1), lambda i, j: (i, 0)),
  ]
  args = [a, w, dinv2]
  if normalize:
    in_specs += [
        pl.BlockSpec((8, k_), lambda i, j: (0, 0)),
        pl.BlockSpec((1, k_), lambda i, j: (0, 0)),
        pl.BlockSpec((1, k_), lambda i, j: (0, 0)),
    ]
    args += [stats, gamma[None, :], beta[None, :]]
  return pl.pallas_call(
      body, grid=(m_ // MB, f_ // nbk), in_specs=in_specs,
      out_specs=pl.BlockSpec((MB, nbk), lambda i, j: (i, j)),
      out_shape=jax.ShapeDtypeStruct((m_, f_), jnp.float32),
  )(*args)


def _t_stats(z, y, dinv2, b):
  f_ = y.shape[1]

  def body(z0_ref, z1_ref, y_ref, d_ref, b_ref, t_ref, s_ref):
    i = pl.program_id(0)
    t = (z0_ref[0] + z1_ref[0] + y_ref[...]) * d_ref[...] + b_ref[...]
    t_ref[...] = t
    rows = i * MB + lax.broadcasted_iota(jnp.int32, (MB, 1), 0)
    tm = jnp.where(rows < NN, t, 0.0)

    @pl.when(i == 0)
    def _():
      s_ref[...] = jnp.zeros_like(s_ref)
    s_ref[0:1, :] += jnp.sum(tm, axis=0, keepdims=True)
    s_ref[1:2, :] += jnp.sum(tm * tm, axis=0, keepdims=True)

  return pl.pallas_call(
      body, grid=(NPAD // MB,),
      in_specs=[
          pl.BlockSpec((1, MB, f_), lambda i: (0, i, 0)),
          pl.BlockSpec((1, MB, f_), lambda i: (1, i, 0)),
          pl.BlockSpec((MB, f_), lambda i: (i, 0)),
          pl.BlockSpec((MB, 1), lambda i: (i, 0)),
          pl.BlockSpec((1, f_), lambda i: (0, 0)),
      ],
      out_specs=[
          pl.BlockSpec((MB, f_), lambda i: (i, 0)),
          pl.BlockSpec((8, f_), lambda i: (0, 0)),
      ],
      out_shape=[
          jax.ShapeDtypeStruct((NPAD, f_), jnp.float32),
          jax.ShapeDtypeStruct((8, f_), jnp.float32),
      ],
  )(z, z, y, dinv2, b[None, :])


def _h3_final(z, y, dinv2, b):
  f_ = y.shape[1]

  def body(z0_ref, z1_ref, y_ref, d_ref, b_ref, o_ref):
    t = (z0_ref[0] + z1_ref[0] + y_ref[...]) * d_ref[...] + b_ref[...]
    o_ref[...] = jnp.maximum(t, 0.0)

  return pl.pallas_call(
      body, grid=(NPAD // MB,),
      in_specs=[
          pl.BlockSpec((1, MB, f_), lambda i: (0, i, 0)),
          pl.BlockSpec((1, MB, f_), lambda i: (1, i, 0)),
          pl.BlockSpec((MB, f_), lambda i: (i, 0)),
          pl.BlockSpec((MB, 1), lambda i: (i, 0)),
          pl.BlockSpec((1, f_), lambda i: (0, 0)),
      ],
      out_specs=pl.BlockSpec((MB, f_), lambda i: (i, 0)),
      out_shape=jax.ShapeDtypeStruct((NPAD, f_), jnp.float32),
  )(z, z, y, dinv2, b[None, :])


def _pool_fc(h3, batch2d, wfc, bfc):
  f_ = h3.shape[1]
  nsteps = NPAD // MB

  def body(h_ref, bt_ref, w_ref, b_ref, o_ref, sums_ref, cnt_ref):
    i = pl.program_id(0)

    @pl.when(i == 0)
    def _():
      sums_ref[...] = jnp.zeros_like(sums_ref)
      cnt_ref[...] = jnp.zeros_like(cnt_ref)
    oh = (bt_ref[...] == lax.broadcasted_iota(jnp.int32, (1, GG), 1)
          ).astype(jnp.float32)
    sums_ref[...] += lax.dot_general(
        oh, h_ref[...], (((0,), (0,)), ((), ())),
        preferred_element_type=jnp.float32)
    cnt_ref[...] += lax.dot_general(
        oh, jnp.ones((MB, 128), jnp.float32), (((0,), (0,)), ((), ())),
        preferred_element_type=jnp.float32)

    @pl.when(i == nsteps - 1)
    def _():
      pooled = sums_ref[...] / jnp.maximum(cnt_ref[:, 0:1], 1.0)
      o_ref[...] = jnp.dot(
          pooled, w_ref[...], preferred_element_type=jnp.float32) + b_ref[...]

  return pl.pallas_call(
      body, grid=(nsteps,),
      in_specs=[
          pl.BlockSpec((MB, f_), lambda i: (i, 0)),
          pl.BlockSpec((MB, 1), lambda i: (i, 0)),
          pl.BlockSpec((f_, OUT), lambda i: (0, 0)),
          pl.BlockSpec((1, OUT), lambda i: (0, 0)),
      ],
      out_specs=pl.BlockSpec((GG, OUT), lambda i: (0, 0)),
      out_shape=jax.ShapeDtypeStruct((GG, OUT), jnp.float32),
      scratch_shapes=[
          pltpu.VMEM((GG, f_), jnp.float32),
          pltpu.VMEM((GG, 128), jnp.float32),
      ],
  )(h3, batch2d, wfc, bfc[None, :])


def kernel(x, edge_index, edge_attr, batch, W1, b1, g1, be1,
           W2, b2, g2, be2, W3, b3, Wfc, bfc):
  x_pad = jnp.pad(x, ((0, NPAD - NN), (0, 0)))
  batch2d = jnp.pad(batch, (0, NPAD - NN), constant_values=GG)[:, None]
  rowp = jnp.pad(edge_index[0], (0, EPAD - EE))
  colp = jnp.pad(edge_index[1], (0, EPAD - EE))
  ewp = jnp.pad(edge_attr, (0, EPAD - EE))
  col_t = colp.reshape(NTILES, NB, KB)
  col_f = colp.reshape(NTILES, EPT)
  ew_t = ewp.reshape(NTILES, EPT)
  row32 = (rowp * 32).reshape(NTILES, EPT)
  row16 = (rowp * 16).reshape(NTILES, EPT)

  degp = _make_deg()(col_f, ew_t)
  dinv2 = _compute_dinv(degp.T)

  y1 = _mm_y(x_pad, W1, dinv2)
  z1 = _spmm_apply(32, 32, row32, col_t, ew_t, y1.reshape(NPAD * 32, 32))
  t1, s1 = _t_stats(z1, y1, dinv2, b1)

  y2 = _mm_y(t1, W2, dinv2, s1, g1, be1)
  z2 = _spmm_apply(32, 32, row32, col_t, ew_t, y2.reshape(NPAD * 32, 32))
  t2, s2 = _t_stats(z2, y2, dinv2, b2)

  y3 = _mm_y(t2, W3, dinv2, s2, g2, be2)
  z3 = _spmm_apply(16, 32, row16, col_t, ew_t, y3.reshape(NPAD * 16, 32))
  h3 = _h3_final(z3, y3, dinv2, b3)

  return _pool_fc(h3, batch2d, Wfc, bfc)


# speedup vs baseline: 1.1920x; 1.0261x over previous
import functools

import jax
import jax.numpy as jnp
from jax import lax
from jax.experimental import pallas as pl
from jax.experimental.pallas import tpu as pltpu
from jax.experimental.pallas import tpu_sc as plsc

NN = 10000
EE = 160000
GG = 64
OUT = 128

NPAD = 10240
NTILES = 32
EPT = 5120
EPAD = EPT * NTILES
KB = 128
NB = EPT // KB
FC = 64
RPT = NPAD // 16
MB = 512

def _mesh():
  return plsc.VectorSubcoreMesh(
      core_axis_name="c", subcore_axis_name="s", num_cores=2, num_subcores=16
  )


@functools.cache
def _make_spmm(nchunk, fc):
  out_t = jax.ShapeDtypeStruct((2, NPAD, nchunk * fc), jnp.float32)
  scratch = [
      pltpu.VMEM((EPT,), jnp.int32),
      pltpu.VMEM((NB, KB), jnp.int32),
      pltpu.VMEM((EPT,), jnp.float32),
      pltpu.VMEM((EPT,), jnp.int32),
      pltpu.VMEM((2, KB, fc), jnp.float32),
      pltpu.VMEM((KB, fc), jnp.float32),
      pltpu.VMEM((KB, fc), jnp.float32),
      pltpu.VMEM_SHARED((NPAD, fc), jnp.float32),
      pltpu.SemaphoreType.DMA,
      pltpu.SemaphoreType.DMA,
  ]
  NG = NB // 2

  @functools.partial(
      pl.kernel, out_type=out_t, mesh=_mesh(), scratch_types=scratch,
      compiler_params=pltpu.CompilerParams(use_tc_tiling_on_sc=False))
  def spmm(row_hbm, col_hbm, ew_hbm, y_hbm, z_hbm,
           rowb, colb, ewb, rowadj, gbufs, sbuf, zbuf, acc, sem0, sem1):
    core = lax.axis_index("c")
    sid = lax.axis_index("s")
    wid = sid * 2 + core
    sems = (sem0, sem1)
    pltpu.sync_copy(row_hbm.at[wid], rowb)
    pltpu.sync_copy(col_hbm.at[wid], colb)
    pltpu.sync_copy(ew_hbm.at[wid], ewb)

    def zero_zbuf(i, _):
      for v in range(fc // 16):
        zbuf[i, pl.ds(v * 16, 16)] = jnp.zeros((16,), jnp.float32)
      return 0
    lax.fori_loop(0, KB, zero_zbuf, 0)

    def issue_gather(b, r):
      pltpu.async_copy(
          y_hbm.at[rowadj.at[pl.ds(b * KB, KB)]], gbufs.at[r], sems[r])

    def wait_gather(r):
      pltpu.make_async_copy(
          y_hbm.at[pl.ds(0, KB)], gbufs.at[r], sems[r]).wait()

    def process(b, r):
      wait_gather(r)

      def scale(s, _):
        ew16 = ewb[pl.ds(b * KB + s * 16, 16)]
        for j in range(16):
          w = ew16[j]
          e = s * 16 + j
          for v in range(fc // 16):
            sbuf[e, pl.ds(v * 16, 16)] = gbufs[r, e, pl.ds(v * 16, 16)] * w
        return 0
      lax.fori_loop(0, KB // 16, scale, 0)
      pltpu.sync_copy(sbuf, acc.at[colb.at[b]], add=True)

    def chunk_body(c, _):
      def adj(i, _):
        rowadj[pl.ds(i * 16, 16)] = rowb[pl.ds(i * 16, 16)] + c
        return 0
      lax.fori_loop(0, EPT // 16, adj, 0)
      for i in range(RPT // KB):
        pltpu.sync_copy(zbuf, acc.at[pl.ds(sid * RPT + i * KB, KB)])
      plsc.subcore_barrier()

      issue_gather(0, 0)
      issue_gather(1, 1)

      def group_body(i, _):
        for r in range(2):
          b = i * 2 + r
          process(b, r)
          issue_gather(b + 2, r)
        return 0
      lax.fori_loop(0, NG - 1, group_body, 0)
      process(NB - 2, 0)
      process(NB - 1, 1)
      plsc.subcore_barrier()
      pltpu.sync_copy(acc.at[pl.ds(sid * RPT, RPT)],
                      z_hbm.at[core, pl.ds(sid * RPT, RPT),
                               pl.ds(c * fc, fc)])
      return 0
    lax.fori_loop(0, nchunk, chunk_body, 0)

  return spmm


def _spmm_apply(nchunk, fc, row_t, col_t, ew_t, ytab):
  return _make_spmm(nchunk, fc)(row_t, col_t, ew_t, ytab)


@functools.cache
def _make_deg():
  out_t = jax.ShapeDtypeStruct((NTILES, NPAD), jnp.float32)
  scratch = [
      pltpu.VMEM((EPT,), jnp.int32),
      pltpu.VMEM((EPT,), jnp.float32),
      pltpu.VMEM((NPAD,), jnp.float32),
  ]

  @functools.partial(
      pl.kernel, out_type=out_t, mesh=_mesh(), scratch_types=scratch,
      compiler_params=pltpu.CompilerParams(
          use_tc_tiling_on_sc=False, needs_layout_passes=False))
  def deg(col_hbm, ew_hbm, out_hbm, colb, ewb, dacc):
    core = lax.axis_index("c")
    sid = lax.axis_index("s")
    wid = sid * 2 + core
    pltpu.sync_copy(col_hbm.at[wid], colb)
    pltpu.sync_copy(ew_hbm.at[wid], ewb)

    def zero(i, _):
      dacc[pl.ds(i * 16, 16)] = jnp.zeros((16,), jnp.float32)
      return 0
    lax.fori_loop(0, NPAD // 16, zero, 0)

    def accum(i, _):
      col16 = colb[pl.ds(i * 16, 16)]
      ew16 = ewb[pl.ds(i * 16, 16)]
      plsc.addupdate_scatter(dacc, [col16], ew16)
      return 0
    lax.fori_loop(0, EPT // 16, accum, 0)
    pltpu.sync_copy(dacc, out_hbm.at[wid])

  return deg


def _compute_dinv(degpt):
  def body(p_ref, o_ref):
    i = pl.program_id(0)
    rows = i * MB + lax.broadcasted_iota(jnp.int32, (MB, 1), 0)
    deg = jnp.sum(p_ref[...], axis=1, keepdims=True) + 1.0
    o_ref[...] = jnp.where(rows < NN, lax.rsqrt(deg), 0.0)

  return pl.pallas_call(
      body, grid=(NPAD // MB,),
      in_specs=[pl.BlockSpec((MB, NTILES), lambda i: (i, 0))],
      out_specs=pl.BlockSpec((MB, 1), lambda i: (i, 0)),
      out_shape=jax.ShapeDtypeStruct((NPAD, 1), jnp.float32),
  )(degpt)


def _mm_y(a, w, dinv2, stats=None, gamma=None, beta=None):
  m_, k_ = a.shape
  f_ = w.shape[1]
  nbk = min(512, f_)
  normalize = stats is not None

  def body(*refs):
    if normalize:
      a_ref, w_ref, d_ref, s_ref, g_ref, be_ref, o_ref = refs
    else:
      a_ref, w_ref, d_ref, o_ref = refs
    aa = a_ref[...]
    if normalize:
      s = s_ref[...]
      mu = s[0:1, :] * (1.0 / NN)
      var = s[1:2, :] * (1.0 / NN) - mu * mu
      aa = jnp.maximum(
          (aa - mu) * lax.rsqrt(var + 1e-5) * g_ref[...] + be_ref[...], 0.0)
    y = jnp.dot(aa, w_ref[...], preferred_element_type=jnp.float32)
    o_ref[...] = y * d_ref[...]

  in_specs = [
      pl.BlockSpec((MB, k_), lambda i, j: (i, 0)),
      pl.BlockSpec((k_, nbk), lambda i, j: (0, j)),
      pl.BlockSpec((MB, 1), lambda i, j: (i, 0)),
  ]
  args = [a, w, dinv2]
  if normalize:
    in_specs += [
        pl.BlockSpec((8, k_), lambda i, j: (0, 0)),
        pl.BlockSpec((1, k_), lambda i, j: (0, 0)),
        pl.BlockSpec((1, k_), lambda i, j: (0, 0)),
    ]
    args += [stats, gamma[None, :], beta[None, :]]
  return pl.pallas_call(
      body, grid=(m_ // MB, f_ // nbk), in_specs=in_specs,
      out_specs=pl.BlockSpec((MB, nbk), lambda i, j: (i, j)),
      out_shape=jax.ShapeDtypeStruct((m_, f_), jnp.float32),
  )(*args)


def _t_stats(z, y, dinv2, b):
  f_ = y.shape[1]

  def body(z0_ref, z1_ref, y_ref, d_ref, b_ref, t_ref, s_ref):
    i = pl.program_id(0)
    t = (z0_ref[0] + z1_ref[0] + y_ref[...]) * d_ref[...] + b_ref[...]
    t_ref[...] = t
    rows = i * MB + lax.broadcasted_iota(jnp.int32, (MB, 1), 0)
    tm = jnp.where(rows < NN, t, 0.0)

    @pl.when(i == 0)
    def _():
      s_ref[...] = jnp.zeros_like(s_ref)
    s_ref[0:1, :] += jnp.sum(tm, axis=0, keepdims=True)
    s_ref[1:2, :] += jnp.sum(tm * tm, axis=0, keepdims=True)

  return pl.pallas_call(
      body, grid=(NPAD // MB,),
      in_specs=[
          pl.BlockSpec((1, MB, f_), lambda i: (0, i, 0)),
          pl.BlockSpec((1, MB, f_), lambda i: (1, i, 0)),
          pl.BlockSpec((MB, f_), lambda i: (i, 0)),
          pl.BlockSpec((MB, 1), lambda i: (i, 0)),
          pl.BlockSpec((1, f_), lambda i: (0, 0)),
      ],
      out_specs=[
          pl.BlockSpec((MB, f_), lambda i: (i, 0)),
          pl.BlockSpec((8, f_), lambda i: (0, 0)),
      ],
      out_shape=[
          jax.ShapeDtypeStruct((NPAD, f_), jnp.float32),
          jax.ShapeDtypeStruct((8, f_), jnp.float32),
      ],
  )(z, z, y, dinv2, b[None, :])


def _h3_final(z, y, dinv2, b):
  f_ = y.shape[1]

  def body(z0_ref, z1_ref, y_ref, d_ref, b_ref, o_ref):
    t = (z0_ref[0] + z1_ref[0] + y_ref[...]) * d_ref[...] + b_ref[...]
    o_ref[...] = jnp.maximum(t, 0.0)

  return pl.pallas_call(
      body, grid=(NPAD // MB,),
      in_specs=[
          pl.BlockSpec((1, MB, f_), lambda i: (0, i, 0)),
          pl.BlockSpec((1, MB, f_), lambda i: (1, i, 0)),
          pl.BlockSpec((MB, f_), lambda i: (i, 0)),
          pl.BlockSpec((MB, 1), lambda i: (i, 0)),
          pl.BlockSpec((1, f_), lambda i: (0, 0)),
      ],
      out_specs=pl.BlockSpec((MB, f_), lambda i: (i, 0)),
      out_shape=jax.ShapeDtypeStruct((NPAD, f_), jnp.float32),
  )(z, z, y, dinv2, b[None, :])


def _pool_fc(h3, batch2d, wfc, bfc):
  f_ = h3.shape[1]
  nsteps = NPAD // MB

  def body(h_ref, bt_ref, w_ref, b_ref, o_ref, sums_ref, cnt_ref):
    i = pl.program_id(0)

    @pl.when(i == 0)
    def _():
      sums_ref[...] = jnp.zeros_like(sums_ref)
      cnt_ref[...] = jnp.zeros_like(cnt_ref)
    oh = (bt_ref[...] == lax.broadcasted_iota(jnp.int32, (1, GG), 1)
          ).astype(jnp.float32)
    sums_ref[...] += lax.dot_general(
        oh, h_ref[...], (((0,), (0,)), ((), ())),
        preferred_element_type=jnp.float32)
    cnt_ref[...] += lax.dot_general(
        oh, jnp.ones((MB, 128), jnp.float32), (((0,), (0,)), ((), ())),
        preferred_element_type=jnp.float32)

    @pl.when(i == nsteps - 1)
    def _():
      pooled = sums_ref[...] / jnp.maximum(cnt_ref[:, 0:1], 1.0)
      o_ref[...] = jnp.dot(
          pooled, w_ref[...], preferred_element_type=jnp.float32) + b_ref[...]

  return pl.pallas_call(
      body, grid=(nsteps,),
      in_specs=[
          pl.BlockSpec((MB, f_), lambda i: (i, 0)),
          pl.BlockSpec((MB, 1), lambda i: (i, 0)),
          pl.BlockSpec((f_, OUT), lambda i: (0, 0)),
          pl.BlockSpec((1, OUT), lambda i: (0, 0)),
      ],
      out_specs=pl.BlockSpec((GG, OUT), lambda i: (0, 0)),
      out_shape=jax.ShapeDtypeStruct((GG, OUT), jnp.float32),
      scratch_shapes=[
          pltpu.VMEM((GG, f_), jnp.float32),
          pltpu.VMEM((GG, 128), jnp.float32),
      ],
  )(h3, batch2d, wfc, bfc[None, :])


def kernel(x, edge_index, edge_attr, batch, W1, b1, g1, be1,
           W2, b2, g2, be2, W3, b3, Wfc, bfc):
  x_pad = jnp.pad(x, ((0, NPAD - NN), (0, 0)))
  batch2d = jnp.pad(batch, (0, NPAD - NN), constant_values=GG)[:, None]
  rowp = jnp.pad(edge_index[0], (0, EPAD - EE))
  colp = jnp.pad(edge_index[1], (0, EPAD - EE))
  ewp = jnp.pad(edge_attr, (0, EPAD - EE))
  col_t = colp.reshape(NTILES, NB, KB)
  col_f = colp.reshape(NTILES, EPT)
  ew_t = ewp.reshape(NTILES, EPT)
  row16 = (rowp * 16).reshape(NTILES, EPT)
  row8 = (rowp * 8).reshape(NTILES, EPT)

  degp = _make_deg()(col_f, ew_t)
  dinv2 = _compute_dinv(degp.T)

  y1 = _mm_y(x_pad, W1, dinv2)
  z1 = _spmm_apply(16, 64, row16, col_t, ew_t, y1.reshape(NPAD * 16, 64))
  t1, s1 = _t_stats(z1, y1, dinv2, b1)

  y2 = _mm_y(t1, W2, dinv2, s1, g1, be1)
  z2 = _spmm_apply(16, 64, row16, col_t, ew_t, y2.reshape(NPAD * 16, 64))
  t2, s2 = _t_stats(z2, y2, dinv2, b2)

  y3 = _mm_y(t2, W3, dinv2, s2, g2, be2)
  z3 = _spmm_apply(8, 64, row8, col_t, ew_t, y3.reshape(NPAD * 8, 64))
  h3 = _h3_final(z3, y3, dinv2, b3)

  return _pool_fc(h3, batch2d, Wfc, bfc)


# speedup vs baseline: 1.1951x; 1.0026x over previous
import functools

import jax
import jax.numpy as jnp
from jax import lax
from jax.experimental import pallas as pl
from jax.experimental.pallas import tpu as pltpu
from jax.experimental.pallas import tpu_sc as plsc

NN = 10000
EE = 160000
GG = 64
OUT = 128

NPAD = 10240
NTILES = 32
EPT = 5120
EPAD = EPT * NTILES
KB = 128
NB = EPT // KB
FC = 64
RPT = NPAD // 16
MB = 512

def _mesh():
  return plsc.VectorSubcoreMesh(
      core_axis_name="c", subcore_axis_name="s", num_cores=2, num_subcores=16
  )


@functools.cache
def _make_spmm(nchunk, fc):
  out_t = jax.ShapeDtypeStruct((2, NPAD, nchunk * fc), jnp.float32)
  scratch = [
      pltpu.VMEM((EPT,), jnp.int32),
      pltpu.VMEM((NB, KB), jnp.int32),
      pltpu.VMEM((EPT,), jnp.float32),
      pltpu.VMEM((EPT,), jnp.int32),
      pltpu.VMEM((2, KB, fc), jnp.float32),
      pltpu.VMEM((2, KB, fc), jnp.float32),
      pltpu.VMEM((KB, fc), jnp.float32),
      pltpu.VMEM_SHARED((NPAD, fc), jnp.float32),
      pltpu.SemaphoreType.DMA,
      pltpu.SemaphoreType.DMA,
      pltpu.SemaphoreType.DMA,
      pltpu.SemaphoreType.DMA,
  ]
  NG = NB // 2

  @functools.partial(
      pl.kernel, out_type=out_t, mesh=_mesh(), scratch_types=scratch,
      compiler_params=pltpu.CompilerParams(use_tc_tiling_on_sc=False))
  def spmm(row_hbm, col_hbm, ew_hbm, y_hbm, z_hbm,
           rowb, colb, ewb, rowadj, gbufs, sbufs, zbuf, acc,
           sem0, sem1, sem2, sem3):
    core = lax.axis_index("c")
    sid = lax.axis_index("s")
    wid = sid * 2 + core
    sems = (sem0, sem1)
    ssems = (sem2, sem3)
    pltpu.sync_copy(row_hbm.at[wid], rowb)
    pltpu.sync_copy(col_hbm.at[wid], colb)
    pltpu.sync_copy(ew_hbm.at[wid], ewb)

    def zero_zbuf(i, _):
      for v in range(fc // 16):
        zbuf[i, pl.ds(v * 16, 16)] = jnp.zeros((16,), jnp.float32)
      return 0
    lax.fori_loop(0, KB, zero_zbuf, 0)

    def issue_gather(b, r):
      pltpu.async_copy(
          y_hbm.at[rowadj.at[pl.ds(b * KB, KB)]], gbufs.at[r], sems[r])

    def wait_gather(r):
      pltpu.make_async_copy(
          y_hbm.at[pl.ds(0, KB)], gbufs.at[r], sems[r]).wait()

    def wait_scatter(r):
      pltpu.make_async_copy(
          y_hbm.at[pl.ds(0, KB)], sbufs.at[r], ssems[r]).wait()

    def process(b, r, first):
      wait_gather(r)
      if not first:
        wait_scatter(r)

      def scale(s, _):
        ew16 = ewb[pl.ds(b * KB + s * 16, 16)]
        for j in range(16):
          w = ew16[j]
          e = s * 16 + j
          for v in range(fc // 16):
            sbufs[r, e, pl.ds(v * 16, 16)] = (
                gbufs[r, e, pl.ds(v * 16, 16)] * w)
        return 0
      lax.fori_loop(0, KB // 16, scale, 0)
      pltpu.async_copy(sbufs.at[r], acc.at[colb.at[b]], ssems[r], add=True)

    def chunk_body(c, _):
      def adj(i, _):
        rowadj[pl.ds(i * 16, 16)] = rowb[pl.ds(i * 16, 16)] + c
        return 0
      lax.fori_loop(0, EPT // 16, adj, 0)
      for i in range(RPT // KB):
        pltpu.sync_copy(zbuf, acc.at[pl.ds(sid * RPT + i * KB, KB)])
      plsc.subcore_barrier()

      issue_gather(0, 0)
      issue_gather(1, 1)
      for r in range(2):
        process(r, r, True)
        issue_gather(r + 2, r)

      def group_body(i, _):
        for r in range(2):
          b = i * 2 + r
          process(b, r, False)
          issue_gather(b + 2, r)
        return 0
      lax.fori_loop(1, NG - 1, group_body, 0)
      for r in range(2):
        process(NB - 2 + r, r, False)
      for r in range(2):
        wait_scatter(r)
      plsc.subcore_barrier()
      pltpu.sync_copy(acc.at[pl.ds(sid * RPT, RPT)],
                      z_hbm.at[core, pl.ds(sid * RPT, RPT),
                               pl.ds(c * fc, fc)])
      return 0
    lax.fori_loop(0, nchunk, chunk_body, 0)

  return spmm


def _spmm_apply(nchunk, fc, row_t, col_t, ew_t, ytab):
  return _make_spmm(nchunk, fc)(row_t, col_t, ew_t, ytab)


@functools.cache
def _make_deg():
  out_t = jax.ShapeDtypeStruct((NTILES, NPAD), jnp.float32)
  scratch = [
      pltpu.VMEM((EPT,), jnp.int32),
      pltpu.VMEM((EPT,), jnp.float32),
      pltpu.VMEM((NPAD,), jnp.float32),
  ]

  @functools.partial(
      pl.kernel, out_type=out_t, mesh=_mesh(), scratch_types=scratch,
      compiler_params=pltpu.CompilerParams(
          use_tc_tiling_on_sc=False, needs_layout_passes=False))
  def deg(col_hbm, ew_hbm, out_hbm, colb, ewb, dacc):
    core = lax.axis_index("c")
    sid = lax.axis_index("s")
    wid = sid * 2 + core
    pltpu.sync_copy(col_hbm.at[wid], colb)
    pltpu.sync_copy(ew_hbm.at[wid], ewb)

    def zero(i, _):
      dacc[pl.ds(i * 16, 16)] = jnp.zeros((16,), jnp.float32)
      return 0
    lax.fori_loop(0, NPAD // 16, zero, 0)

    def accum(i, _):
      col16 = colb[pl.ds(i * 16, 16)]
      ew16 = ewb[pl.ds(i * 16, 16)]
      plsc.addupdate_scatter(dacc, [col16], ew16)
      return 0
    lax.fori_loop(0, EPT // 16, accum, 0)
    pltpu.sync_copy(dacc, out_hbm.at[wid])

  return deg


def _compute_dinv(degpt):
  def body(p_ref, o_ref):
    i = pl.program_id(0)
    rows = i * MB + lax.broadcasted_iota(jnp.int32, (MB, 1), 0)
    deg = jnp.sum(p_ref[...], axis=1, keepdims=True) + 1.0
    o_ref[...] = jnp.where(rows < NN, lax.rsqrt(deg), 0.0)

  return pl.pallas_call(
      body, grid=(NPAD // MB,),
      in_specs=[pl.BlockSpec((MB, NTILES), lambda i: (i, 0))],
      out_specs=pl.BlockSpec((MB, 1), lambda i: (i, 0)),
      out_shape=jax.ShapeDtypeStruct((NPAD, 1), jnp.float32),
  )(degpt)


def _mm_y(a, w, dinv2, stats=None, gamma=None, beta=None):
  m_, k_ = a.shape
  f_ = w.shape[1]
  nbk = min(512, f_)
  normalize = stats is not None

  def body(*refs):
    if normalize:
      a_ref, w_ref, d_ref, s_ref, g_ref, be_ref, o_ref = refs
    else:
      a_ref, w_ref, d_ref, o_ref = refs
    aa = a_ref[...]
    if normalize:
      s = s_ref[...]
      mu = s[0:1, :] * (1.0 / NN)
      var = s[1:2, :] * (1.0 / NN) - mu * mu
      aa = jnp.maximum(
          (aa - mu) * lax.rsqrt(var + 1e-5) * g_ref[...] + be_ref[...], 0.0)
    y = jnp.dot(aa, w_ref[...], preferred_element_type=jnp.float32)
    o_ref[...] = y * d_ref[...]

  in_specs = [
      pl.BlockSpec((MB, k_), lambda i, j: (i, 0)),
      pl.BlockSpec((k_, nbk), lambda i, j: (0, j)),
      pl.BlockSpec((MB, 1), lambda i, j: (i, 0)),
  ]
  args = [a, w, dinv2]
  if normalize:
    in_specs += [
        pl.BlockSpec((8, k_), lambda i, j: (0, 0)),
        pl.BlockSpec((1, k_), lambda i, j: (0, 0)),
        pl.BlockSpec((1, k_), lambda i, j: (0, 0)),
    ]
    args += [stats, gamma[None, :], beta[None, :]]
  return pl.pallas_call(
      body, grid=(m_ // MB, f_ // nbk), in_specs=in_specs,
      out_specs=pl.BlockSpec((MB, nbk), lambda i, j: (i, j)),
      out_shape=jax.ShapeDtypeStruct((m_, f_), jnp.float32),
  )(*args)


def _t_stats(z, y, dinv2, b):
  f_ = y.shape[1]

  def body(z0_ref, z1_ref, y_ref, d_ref, b_ref, t_ref, s_ref):
    i = pl.program_id(0)
    t = (z0_ref[0] + z1_ref[0] + y_ref[...]) * d_ref[...] + b_ref[...]
    t_ref[...] = t
    rows = i * MB + lax.broadcasted_iota(jnp.int32, (MB, 1), 0)
    tm = jnp.where(rows < NN, t, 0.0)

    @pl.when(i == 0)
    def _():
      s_ref[...] = jnp.zeros_like(s_ref)
    s_ref[0:1, :] += jnp.sum(tm, axis=0, keepdims=True)
    s_ref[1:2, :] += jnp.sum(tm * tm, axis=0, keepdims=True)

  return pl.pallas_call(
      body, grid=(NPAD // MB,),
      in_specs=[
          pl.BlockSpec((1, MB, f_), lambda i: (0, i, 0)),
          pl.BlockSpec((1, MB, f_), lambda i: (1, i, 0)),
          pl.BlockSpec((MB, f_), lambda i: (i, 0)),
          pl.BlockSpec((MB, 1), lambda i: (i, 0)),
          pl.BlockSpec((1, f_), lambda i: (0, 0)),
      ],
      out_specs=[
          pl.BlockSpec((MB, f_), lambda i: (i, 0)),
          pl.BlockSpec((8, f_), lambda i: (0, 0)),
      ],
      out_shape=[
          jax.ShapeDtypeStruct((NPAD, f_), jnp.float32),
          jax.ShapeDtypeStruct((8, f_), jnp.float32),
      ],
  )(z, z, y, dinv2, b[None, :])


def _h3_final(z, y, dinv2, b):
  f_ = y.shape[1]

  def body(z0_ref, z1_ref, y_ref, d_ref, b_ref, o_ref):
    t = (z0_ref[0] + z1_ref[0] + y_ref[...]) * d_ref[...] + b_ref[...]
    o_ref[...] = jnp.maximum(t, 0.0)

  return pl.pallas_call(
      body, grid=(NPAD // MB,),
      in_specs=[
          pl.BlockSpec((1, MB, f_), lambda i: (0, i, 0)),
          pl.BlockSpec((1, MB, f_), lambda i: (1, i, 0)),
          pl.BlockSpec((MB, f_), lambda i: (i, 0)),
          pl.BlockSpec((MB, 1), lambda i: (i, 0)),
          pl.BlockSpec((1, f_), lambda i: (0, 0)),
      ],
      out_specs=pl.BlockSpec((MB, f_), lambda i: (i, 0)),
      out_shape=jax.ShapeDtypeStruct((NPAD, f_), jnp.float32),
  )(z, z, y, dinv2, b[None, :])


def _pool_fc(h3, batch2d, wfc, bfc):
  f_ = h3.shape[1]
  nsteps = NPAD // MB

  def body(h_ref, bt_ref, w_ref, b_ref, o_ref, sums_ref, cnt_ref):
    i = pl.program_id(0)

    @pl.when(i == 0)
    def _():
      sums_ref[...] = jnp.zeros_like(sums_ref)
      cnt_ref[...] = jnp.zeros_like(cnt_ref)
    oh = (bt_ref[...] == lax.broadcasted_iota(jnp.int32, (1, GG), 1)
          ).astype(jnp.float32)
    sums_ref[...] += lax.dot_general(
        oh, h_ref[...], (((0,), (0,)), ((), ())),
        preferred_element_type=jnp.float32)
    cnt_ref[...] += lax.dot_general(
        oh, jnp.ones((MB, 128), jnp.float32), (((0,), (0,)), ((), ())),
        preferred_element_type=jnp.float32)

    @pl.when(i == nsteps - 1)
    def _():
      pooled = sums_ref[...] / jnp.maximum(cnt_ref[:, 0:1], 1.0)
      o_ref[...] = jnp.dot(
          pooled, w_ref[...], preferred_element_type=jnp.float32) + b_ref[...]

  return pl.pallas_call(
      body, grid=(nsteps,),
      in_specs=[
          pl.BlockSpec((MB, f_), lambda i: (i, 0)),
          pl.BlockSpec((MB, 1), lambda i: (i, 0)),
          pl.BlockSpec((f_, OUT), lambda i: (0, 0)),
          pl.BlockSpec((1, OUT), lambda i: (0, 0)),
      ],
      out_specs=pl.BlockSpec((GG, OUT), lambda i: (0, 0)),
      out_shape=jax.ShapeDtypeStruct((GG, OUT), jnp.float32),
      scratch_shapes=[
          pltpu.VMEM((GG, f_), jnp.float32),
          pltpu.VMEM((GG, 128), jnp.float32),
      ],
  )(h3, batch2d, wfc, bfc[None, :])


def kernel(x, edge_index, edge_attr, batch, W1, b1, g1, be1,
           W2, b2, g2, be2, W3, b3, Wfc, bfc):
  x_pad = jnp.pad(x, ((0, NPAD - NN), (0, 0)))
  batch2d = jnp.pad(batch, (0, NPAD - NN), constant_values=GG)[:, None]
  rowp = jnp.pad(edge_index[0], (0, EPAD - EE))
  colp = jnp.pad(edge_index[1], (0, EPAD - EE))
  ewp = jnp.pad(edge_attr, (0, EPAD - EE))
  col_t = colp.reshape(NTILES, NB, KB)
  col_f = colp.reshape(NTILES, EPT)
  ew_t = ewp.reshape(NTILES, EPT)
  row16 = (rowp * 16).reshape(NTILES, EPT)
  row8 = (rowp * 8).reshape(NTILES, EPT)

  degp = _make_deg()(col_f, ew_t)
  dinv2 = _compute_dinv(degp.T)

  y1 = _mm_y(x_pad, W1, dinv2)
  z1 = _spmm_apply(16, 64, row16, col_t, ew_t, y1.reshape(NPAD * 16, 64))
  t1, s1 = _t_stats(z1, y1, dinv2, b1)

  y2 = _mm_y(t1, W2, dinv2, s1, g1, be1)
  z2 = _spmm_apply(16, 64, row16, col_t, ew_t, y2.reshape(NPAD * 16, 64))
  t2, s2 = _t_stats(z2, y2, dinv2, b2)

  y3 = _mm_y(t2, W3, dinv2, s2, g2, be2)
  z3 = _spmm_apply(8, 64, row8, col_t, ew_t, y3.reshape(NPAD * 8, 64))
  h3 = _h3_final(z3, y3, dinv2, b3)

  return _pool_fc(h3, batch2d, Wfc, bfc)


# speedup vs baseline: 1.1952x; 1.0001x over previous
import functools

import jax
import jax.numpy as jnp
from jax import lax
from jax.experimental import pallas as pl
from jax.experimental.pallas import tpu as pltpu
from jax.experimental.pallas import tpu_sc as plsc

NN = 10000
EE = 160000
GG = 64
OUT = 128

NPAD = 10240
NTILES = 32
EPT = 5120
EPAD = EPT * NTILES
KB = 128
NB = EPT // KB
RPT = NPAD // 16
MB = 512

def _mesh():
  return plsc.VectorSubcoreMesh(
      core_axis_name="c", subcore_axis_name="s", num_cores=2, num_subcores=16
  )


@functools.cache
def _make_spmm(nchunk, fc):
  out_t = jax.ShapeDtypeStruct((2, NPAD, nchunk * fc), jnp.float32)
  scratch = [
      pltpu.VMEM((EPT,), jnp.int32),
      pltpu.VMEM((NB, KB), jnp.int32),
      pltpu.VMEM((EPT,), jnp.float32),
      pltpu.VMEM((EPT,), jnp.int32),
      pltpu.VMEM((2, KB, fc), jnp.float32),
      pltpu.VMEM((2, KB, fc), jnp.float32),
      pltpu.VMEM((KB, fc), jnp.float32),
      pltpu.VMEM_SHARED((NPAD, fc), jnp.float32),
      pltpu.SemaphoreType.DMA,
      pltpu.SemaphoreType.DMA,
      pltpu.SemaphoreType.DMA,
      pltpu.SemaphoreType.DMA,
  ]
  NG = NB // 2

  @functools.partial(
      pl.kernel, out_type=out_t, mesh=_mesh(), scratch_types=scratch,
      compiler_params=pltpu.CompilerParams(use_tc_tiling_on_sc=False))
  def spmm(row_hbm, col_hbm, ew_hbm, y_hbm, z_hbm,
           rowb, colb, ewb, rowadj, gbufs, sbufs, zbuf, acc,
           sem0, sem1, sem2, sem3):
    core = lax.axis_index("c")
    sid = lax.axis_index("s")
    wid = sid * 2 + core
    sems = (sem0, sem1)
    ssems = (sem2, sem3)
    pltpu.sync_copy(row_hbm.at[wid], rowb)
    pltpu.sync_copy(col_hbm.at[wid], colb)
    pltpu.sync_copy(ew_hbm.at[wid], ewb)

    def zero_zbuf(i, _):
      for v in range(fc // 16):
        zbuf[i, pl.ds(v * 16, 16)] = jnp.zeros((16,), jnp.float32)
      return 0
    lax.fori_loop(0, KB, zero_zbuf, 0)

    def issue_gather(b, r):
      pltpu.async_copy(
          y_hbm.at[rowadj.at[pl.ds(b * KB, KB)]], gbufs.at[r], sems[r])

    def wait_gather(r):
      pltpu.make_async_copy(
          y_hbm.at[pl.ds(0, KB)], gbufs.at[r], sems[r]).wait()

    def wait_scatter(r):
      pltpu.make_async_copy(
          y_hbm.at[pl.ds(0, KB)], sbufs.at[r], ssems[r]).wait()

    def process(b, r, first):
      wait_gather(r)
      if not first:
        wait_scatter(r)

      def scale(s, _):
        ew16 = ewb[pl.ds(b * KB + s * 16, 16)]
        for j in range(16):
          w = ew16[j]
          e = s * 16 + j
          for v in range(fc // 16):
            sbufs[r, e, pl.ds(v * 16, 16)] = (
                gbufs[r, e, pl.ds(v * 16, 16)] * w)
        return 0
      lax.fori_loop(0, KB // 16, scale, 0)
      pltpu.async_copy(sbufs.at[r], acc.at[colb.at[b]], ssems[r], add=True)

    def chunk_body(c, _):
      def adj(i, _):
        rowadj[pl.ds(i * 16, 16)] = rowb[pl.ds(i * 16, 16)] + c
        return 0
      lax.fori_loop(0, EPT // 16, adj, 0)
      for i in range(RPT // KB):
        pltpu.sync_copy(zbuf, acc.at[pl.ds(sid * RPT + i * KB, KB)])
      plsc.subcore_barrier()

      issue_gather(0, 0)
      issue_gather(1, 1)
      for r in range(2):
        process(r, r, True)
        issue_gather(r + 2, r)

      def group_body(i, _):
        for r in range(2):
          b = i * 2 + r
          process(b, r, False)
          issue_gather(b + 2, r)
        return 0
      lax.fori_loop(1, NG - 1, group_body, 0)
      for r in range(2):
        process(NB - 2 + r, r, False)
      for r in range(2):
        wait_scatter(r)
      plsc.subcore_barrier()
      pltpu.sync_copy(acc.at[pl.ds(sid * RPT, RPT)],
                      z_hbm.at[core, pl.ds(sid * RPT, RPT),
                               pl.ds(c * fc, fc)])
      return 0
    lax.fori_loop(0, nchunk, chunk_body, 0)

  return spmm


def _spmm_apply(nchunk, fc, row_t, col_t, ew_t, ytab):
  return _make_spmm(nchunk, fc)(row_t, col_t, ew_t, ytab)


@functools.cache
def _make_deg():
  out_t = jax.ShapeDtypeStruct((NTILES, NPAD), jnp.float32)
  scratch = [
      pltpu.VMEM((EPT,), jnp.int32),
      pltpu.VMEM((EPT,), jnp.float32),
      pltpu.VMEM((NPAD,), jnp.float32),
  ]

  @functools.partial(
      pl.kernel, out_type=out_t, mesh=_mesh(), scratch_types=scratch,
      compiler_params=pltpu.CompilerParams(
          use_tc_tiling_on_sc=False, needs_layout_passes=False))
  def deg(col_hbm, ew_hbm, out_hbm, colb, ewb, dacc):
    core = lax.axis_index("c")
    sid = lax.axis_index("s")
    wid = sid * 2 + core
    pltpu.sync_copy(col_hbm.at[wid], colb)
    pltpu.sync_copy(ew_hbm.at[wid], ewb)

    def zero(i, _):
      dacc[pl.ds(i * 16, 16)] = jnp.zeros((16,), jnp.float32)
      return 0
    lax.fori_loop(0, NPAD // 16, zero, 0)

    def accum(i, _):
      col16 = colb[pl.ds(i * 16, 16)]
      ew16 = ewb[pl.ds(i * 16, 16)]
      plsc.addupdate_scatter(dacc, [col16], ew16)
      return 0
    lax.fori_loop(0, EPT // 16, accum, 0)
    pltpu.sync_copy(dacc, out_hbm.at[wid])

  return deg


def _compute_dinv(degpt):
  def body(p_ref, o_ref):
    i = pl.program_id(0)
    rows = i * MB + lax.broadcasted_iota(jnp.int32, (MB, 1), 0)
    deg = jnp.sum(p_ref[...], axis=1, keepdims=True) + 1.0
    o_ref[...] = jnp.where(rows < NN, lax.rsqrt(deg), 0.0)

  return pl.pallas_call(
      body, grid=(NPAD // MB,),
      in_specs=[pl.BlockSpec((MB, NTILES), lambda i: (i, 0))],
      out_specs=pl.BlockSpec((MB, 1), lambda i: (i, 0)),
      out_shape=jax.ShapeDtypeStruct((NPAD, 1), jnp.float32),
  )(degpt)


def _mm_y(a, w, dinv2, stats=None, gamma=None, beta=None):
  m_, k_ = a.shape
  f_ = w.shape[1]
  nbk = min(512, f_)
  normalize = stats is not None

  def body(*refs):
    if normalize:
      a_ref, w_ref, d_ref, s_ref, g_ref, be_ref, o_ref = refs
    else:
      a_ref, w_ref, d_ref, o_ref = refs
    aa = a_ref[...]
    if normalize:
      s = s_ref[...]
      mu = s[0:1, :] * (1.0 / NN)
      var = s[1:2, :] * (1.0 / NN) - mu * mu
      aa = jnp.maximum(
          (aa - mu) * lax.rsqrt(var + 1e-5) * g_ref[...] + be_ref[...], 0.0)
    y = jnp.dot(aa, w_ref[...], preferred_element_type=jnp.float32)
    o_ref[...] = y * d_ref[...]

  in_specs = [
      pl.BlockSpec((MB, k_), lambda i, j: (i, 0)),
      pl.BlockSpec((k_, nbk), lambda i, j: (0, j)),
      pl.BlockSpec((MB, 1), lambda i, j: (i, 0)),
  ]
  args = [a, w, dinv2]
  if normalize:
    in_specs += [
        pl.BlockSpec((8, k_), lambda i, j: (0, 0)),
        pl.BlockSpec((1, k_), lambda i, j: (0, 0)),
        pl.BlockSpec((1, k_), lambda i, j: (0, 0)),
    ]
    args += [stats, gamma[None, :], beta[None, :]]
  return pl.pallas_call(
      body, grid=(m_ // MB, f_ // nbk), in_specs=in_specs,
      out_specs=pl.BlockSpec((MB, nbk), lambda i, j: (i, j)),
      out_shape=jax.ShapeDtypeStruct((m_, f_), jnp.float32),
  )(*args)


def _t_stats(z, y, dinv2, b):
  f_ = y.shape[1]

  def body(z0_ref, z1_ref, y_ref, d_ref, b_ref, t_ref, s_ref):
    i = pl.program_id(0)
    t = (z0_ref[0] + z1_ref[0] + y_ref[...]) * d_ref[...] + b_ref[...]
    t_ref[...] = t
    rows = i * MB + lax.broadcasted_iota(jnp.int32, (MB, 1), 0)
    tm = jnp.where(rows < NN, t, 0.0)

    @pl.when(i == 0)
    def _():
      s_ref[...] = jnp.zeros_like(s_ref)
    s_ref[0:1, :] += jnp.sum(tm, axis=0, keepdims=True)
    s_ref[1:2, :] += jnp.sum(tm * tm, axis=0, keepdims=True)

  return pl.pallas_call(
      body, grid=(NPAD // MB,),
      in_specs=[
          pl.BlockSpec((1, MB, f_), lambda i: (0, i, 0)),
          pl.BlockSpec((1, MB, f_), lambda i: (1, i, 0)),
          pl.BlockSpec((MB, f_), lambda i: (i, 0)),
          pl.BlockSpec((MB, 1), lambda i: (i, 0)),
          pl.BlockSpec((1, f_), lambda i: (0, 0)),
      ],
      out_specs=[
          pl.BlockSpec((MB, f_), lambda i: (i, 0)),
          pl.BlockSpec((8, f_), lambda i: (0, 0)),
      ],
      out_shape=[
          jax.ShapeDtypeStruct((NPAD, f_), jnp.float32),
          jax.ShapeDtypeStruct((8, f_), jnp.float32),
      ],
  )(z, z, y, dinv2, b[None, :])


def _h3_final(z, y, dinv2, b):
  f_ = y.shape[1]

  def body(z0_ref, z1_ref, y_ref, d_ref, b_ref, o_ref):
    t = (z0_ref[0] + z1_ref[0] + y_ref[...]) * d_ref[...] + b_ref[...]
    o_ref[...] = jnp.maximum(t, 0.0)

  return pl.pallas_call(
      body, grid=(NPAD // MB,),
      in_specs=[
          pl.BlockSpec((1, MB, f_), lambda i: (0, i, 0)),
          pl.BlockSpec((1, MB, f_), lambda i: (1, i, 0)),
          pl.BlockSpec((MB, f_), lambda i: (i, 0)),
          pl.BlockSpec((MB, 1), lambda i: (i, 0)),
          pl.BlockSpec((1, f_), lambda i: (0, 0)),
      ],
      out_specs=pl.BlockSpec((MB, f_), lambda i: (i, 0)),
      out_shape=jax.ShapeDtypeStruct((NPAD, f_), jnp.float32),
  )(z, z, y, dinv2, b[None, :])


def _pool_fc(h3, batch2d, wfc, bfc):
  f_ = h3.shape[1]
  nsteps = NPAD // MB

  def body(h_ref, bt_ref, w_ref, b_ref, o_ref, sums_ref, cnt_ref):
    i = pl.program_id(0)

    @pl.when(i == 0)
    def _():
      sums_ref[...] = jnp.zeros_like(sums_ref)
      cnt_ref[...] = jnp.zeros_like(cnt_ref)
    oh = (bt_ref[...] == lax.broadcasted_iota(jnp.int32, (1, GG), 1)
          ).astype(jnp.float32)
    sums_ref[...] += lax.dot_general(
        oh, h_ref[...], (((0,), (0,)), ((), ())),
        preferred_element_type=jnp.float32)
    cnt_ref[...] += lax.dot_general(
        oh, jnp.ones((MB, 128), jnp.float32), (((0,), (0,)), ((), ())),
        preferred_element_type=jnp.float32)

    @pl.when(i == nsteps - 1)
    def _():
      pooled = sums_ref[...] / jnp.maximum(cnt_ref[:, 0:1], 1.0)
      o_ref[...] = jnp.dot(
          pooled, w_ref[...], preferred_element_type=jnp.float32) + b_ref[...]

  return pl.pallas_call(
      body, grid=(nsteps,),
      in_specs=[
          pl.BlockSpec((MB, f_), lambda i: (i, 0)),
          pl.BlockSpec((MB, 1), lambda i: (i, 0)),
          pl.BlockSpec((f_, OUT), lambda i: (0, 0)),
          pl.BlockSpec((1, OUT), lambda i: (0, 0)),
      ],
      out_specs=pl.BlockSpec((GG, OUT), lambda i: (0, 0)),
      out_shape=jax.ShapeDtypeStruct((GG, OUT), jnp.float32),
      scratch_shapes=[
          pltpu.VMEM((GG, f_), jnp.float32),
          pltpu.VMEM((GG, 128), jnp.float32),
      ],
  )(h3, batch2d, wfc, bfc[None, :])


def kernel(x, edge_index, edge_attr, batch, W1, b1, g1, be1,
           W2, b2, g2, be2, W3, b3, Wfc, bfc):
  x_pad = jnp.pad(x, ((0, NPAD - NN), (0, 0)))
  batch2d = jnp.pad(batch, (0, NPAD - NN), constant_values=GG)[:, None]
  rowp = jnp.pad(edge_index[0], (0, EPAD - EE))
  colp = jnp.pad(edge_index[1], (0, EPAD - EE))
  ewp = jnp.pad(edge_attr, (0, EPAD - EE))
  col_t = colp.reshape(NTILES, NB, KB)
  col_f = colp.reshape(NTILES, EPT)
  ew_t = ewp.reshape(NTILES, EPT)
  row16 = (rowp * 16).reshape(NTILES, EPT)
  row8 = (rowp * 8).reshape(NTILES, EPT)

  degp = _make_deg()(col_f, ew_t)
  dinv2 = _compute_dinv(degp.T)

  y1 = _mm_y(x_pad, W1, dinv2)
  z1 = _spmm_apply(16, 64, row16, col_t, ew_t, y1.reshape(NPAD * 16, 64))
  t1, s1 = _t_stats(z1, y1, dinv2, b1)

  y2 = _mm_y(t1, W2, dinv2, s1, g1, be1)
  z2 = _spmm_apply(16, 64, row16, col_t, ew_t, y2.reshape(NPAD * 16, 64))
  t2, s2 = _t_stats(z2, y2, dinv2, b2)

  y3 = _mm_y(t2, W3, dinv2, s2, g2, be2)
  z3 = _spmm_apply(8, 64, row8, col_t, ew_t, y3.reshape(NPAD * 8, 64))
  h3 = _h3_final(z3, y3, dinv2, b3)

  return _pool_fc(h3, batch2d, Wfc, bfc)
